# baseline (speedup 1.0000x reference)
import functools

import jax
import jax.numpy as jnp
from jax import lax
from jax.experimental import pallas as pl
from jax.experimental.pallas import tpu as pltpu
from jax.experimental.pallas import tpu_sc as plsc


def _sc_transe(B, D, n_workers):
    b_per_w = B // n_workers
    n_chunks = b_per_w // 128
    mesh = plsc.VectorSubcoreMesh(core_axis_name="c", subcore_axis_name="s")
    num_cores = 2

    @functools.partial(
        pl.kernel,
        mesh=mesh,
        out_type=jax.ShapeDtypeStruct((B,), jnp.float32),
        compiler_params=pltpu.CompilerParams(
            needs_layout_passes=False, use_tc_tiling_on_sc=False),
        scratch_types=[
            pltpu.VMEM((n_chunks, 128), jnp.int32),
            pltpu.VMEM((n_chunks, 128), jnp.int32),
            pltpu.VMEM((n_chunks, 128), jnp.int32),
            pltpu.VMEM((b_per_w, D), jnp.float32),
            pltpu.VMEM((b_per_w, D), jnp.float32),
            pltpu.VMEM((b_per_w, D), jnp.float32),
            pltpu.VMEM((b_per_w,), jnp.float32),
            pltpu.VMEM((256,), jnp.float32),
            pltpu.SemaphoreType.DMA,
        ],
    )
    def k(heads_hbm, rel_hbm, tails_hbm, ent_hbm, reltab_hbm, out_hbm,
          idx_h, idx_r, idx_t, e1, rbuf, e2, outv, tbuf, sem):
        wid = lax.axis_index("s") * num_cores + lax.axis_index("c")
        base = wid * b_per_w

        for c in range(n_chunks):
            off = base + c * 128
            pltpu.sync_copy(heads_hbm.at[pl.ds(off, 128)], idx_h.at[c])
            pltpu.sync_copy(rel_hbm.at[pl.ds(off, 128)], idx_r.at[c])
            pltpu.sync_copy(tails_hbm.at[pl.ds(off, 128)], idx_t.at[c])

        copies = []
        for c in range(n_chunks):
            rows = pl.ds(c * 128, 128)
            copies.append(pltpu.async_copy(ent_hbm.at[idx_h.at[c]], e1.at[rows], sem))
            copies.append(pltpu.async_copy(reltab_hbm.at[idx_r.at[c]], rbuf.at[rows], sem))
            copies.append(pltpu.async_copy(ent_hbm.at[idx_t.at[c]], e2.at[rows], sem))
        for cp in copies:
            cp.wait()

        lane = lax.iota(jnp.int32, 16)
        lane16 = lane * 16

        def block(b, carry):
            for r in range(16):
                row = b * 16 + r
                s = jnp.zeros((16,), jnp.float32)
                for jj in range(D // 16):
                    cols = pl.ds(jj * 16, 16)
                    d = (e1[row, cols] + rbuf[row, cols]) - e2[row, cols]
                    s = s + d * d
                tbuf[pl.ds(r * 16, 16)] = s
            acc = jnp.zeros((16,), jnp.float32)
            for c in range(16):
                acc = acc + plsc.load_gather(tbuf, [lane16 + c])
            outv[pl.ds(b * 16, 16)] = -acc
            return carry

        lax.fori_loop(0, b_per_w // 16, block, 0)

        pltpu.sync_copy(outv, out_hbm.at[pl.ds(base, b_per_w)])

    return k


def kernel(heads, relations, tails, entity_embedding, relation_embedding):
    B = heads.shape[0]
    D = entity_embedding.shape[1]
    k = _sc_transe(B, D, 32)
    return k(heads, relations, tails, entity_embedding, relation_embedding)


# speedup vs baseline: 1.0317x; 1.0317x over previous
import functools

import jax
import jax.numpy as jnp
from jax import lax
from jax.experimental import pallas as pl
from jax.experimental.pallas import tpu as pltpu
from jax.experimental.pallas import tpu_sc as plsc


def _sc_transe(B, D, n_ent, n_rel, n_workers):
    b_per_w = B // n_workers
    n_pass = 2
    p_items = b_per_w // n_pass
    n_chunks = b_per_w // 128
    w = 128
    mesh = plsc.VectorSubcoreMesh(core_axis_name="c", subcore_axis_name="s")
    num_cores = 2

    @functools.partial(
        pl.kernel,
        mesh=mesh,
        out_type=jax.ShapeDtypeStruct((B,), jnp.float32),
        compiler_params=pltpu.CompilerParams(
            needs_layout_passes=False, use_tc_tiling_on_sc=False),
        scratch_types=[
            pltpu.VMEM((n_chunks, 128), jnp.int32),
            pltpu.VMEM((n_chunks, 128), jnp.int32),
            pltpu.VMEM((n_chunks, 128), jnp.int32),
            pltpu.VMEM((p_items, w), jnp.float32),
            pltpu.VMEM((p_items, w), jnp.float32),
            pltpu.VMEM((p_items, w), jnp.float32),
            pltpu.VMEM((b_per_w,), jnp.float32),
            pltpu.SemaphoreType.DMA,
        ],
    )
    def k(heads_hbm, rel_hbm, tails_hbm, ev_hbm, rv_hbm, out_hbm,
          idx_h, idx_r, idx_t, e1m, erm, e2m, outv, sem):
        wid = lax.axis_index("s") * num_cores + lax.axis_index("c")
        base = wid * b_per_w

        for c in range(n_chunks):
            off = base + c * 128
            pltpu.sync_copy(heads_hbm.at[pl.ds(off, 128)], idx_h.at[c])
            pltpu.sync_copy(rel_hbm.at[pl.ds(off, 128)], idx_r.at[c])
            pltpu.sync_copy(tails_hbm.at[pl.ds(off, 128)], idx_t.at[c])

        lane = lax.iota(jnp.int32, 16)

        for p in range(n_pass):
            copies = []
            for c in range(p_items // 128):
                g = p * (p_items // 128) + c
                rows = pl.ds(c * 128, 128)
                copies.append(pltpu.async_copy(
                    ev_hbm.at[idx_h.at[g]], e1m.at[rows], sem))
                copies.append(pltpu.async_copy(
                    rv_hbm.at[idx_r.at[g]], erm.at[rows], sem))
                copies.append(pltpu.async_copy(
                    ev_hbm.at[idx_t.at[g]], e2m.at[rows], sem))
            for cp in copies:
                cp.wait()

            def block(b, carry):
                row16 = b * 16 + lane
                acc = jnp.zeros((16,), jnp.float32)
                for c in range(D):
                    col = jnp.full((16,), c, jnp.int32)
                    h = plsc.load_gather(e1m, [row16, col])
                    r = plsc.load_gather(erm, [row16, col])
                    t = plsc.load_gather(e2m, [row16, col])
                    d = (h + r) - t
                    acc = acc + d * d
                outv[pl.ds(p * p_items + b * 16, 16)] = -acc
                return carry

            lax.fori_loop(0, p_items // 16, block, 0)

        pltpu.sync_copy(outv, out_hbm.at[pl.ds(base, b_per_w)])

    return k


def kernel(heads, relations, tails, entity_embedding, relation_embedding):
    B = heads.shape[0]
    n_ent, D = entity_embedding.shape
    n_rel = relation_embedding.shape[0]
    ev = jnp.pad(entity_embedding, ((0, 0), (0, 128 - D)))
    rv = jnp.pad(relation_embedding, ((0, 0), (0, 128 - D)))
    k = _sc_transe(B, D, n_ent, n_rel, 32)
    return k(heads, relations, tails, ev, rv)


# speedup vs baseline: 1.3227x; 1.2820x over previous
import functools

import jax
import jax.numpy as jnp
from jax import lax
from jax.experimental import pallas as pl
from jax.experimental.pallas import tpu as pltpu
from jax.experimental.pallas import tpu_sc as plsc


def _sc_transe(B, D, n_workers):
    b_per_w = B // n_workers
    p_items = 32
    n_pass = b_per_w // p_items
    slots = p_items * 8
    mesh = plsc.VectorSubcoreMesh(core_axis_name="c", subcore_axis_name="s")
    num_cores = 2

    @functools.partial(
        pl.kernel,
        mesh=mesh,
        out_type=jax.ShapeDtypeStruct((B,), jnp.float32),
        compiler_params=pltpu.CompilerParams(
            needs_layout_passes=False, use_tc_tiling_on_sc=True),
        scratch_types=[
            pltpu.VMEM((b_per_w,), jnp.int32),
            pltpu.VMEM((b_per_w,), jnp.int32),
            pltpu.VMEM((b_per_w,), jnp.int32),
            pltpu.VMEM((slots, D), jnp.float32),
            pltpu.VMEM((slots, D), jnp.float32),
            pltpu.VMEM((slots, D), jnp.float32),
            pltpu.VMEM((b_per_w,), jnp.float32),
            pltpu.SemaphoreType.DMA,
            pltpu.SemaphoreType.DMA,
            pltpu.SemaphoreType.DMA,
        ],
    )
    def k(heads_hbm, rel_hbm, tails_hbm, ev_hbm, rv_hbm, out_hbm,
          idx_h, idx_r, idx_t, e1m, erm, e2m, outv, sem_h, sem_r, sem_t):
        wid = lax.axis_index("s") * num_cores + lax.axis_index("c")
        base = wid * b_per_w

        pltpu.sync_copy(heads_hbm.at[pl.ds(base, b_per_w)], idx_h)
        pltpu.sync_copy(rel_hbm.at[pl.ds(base, b_per_w)], idx_r)
        pltpu.sync_copy(tails_hbm.at[pl.ds(base, b_per_w)], idx_t)

        lane = lax.iota(jnp.int32, 16)

        def do_pass(p, carry):
            for g in range(p_items // 16):
                off = p * p_items + g * 16
                jh = idx_h[pl.ds(off, 16)] >> 3
                jr = idx_r[pl.ds(off, 16)] >> 3
                jt = idx_t[pl.ds(off, 16)] >> 3
                for l in range(16):
                    slot = g * 16 + l
                    dst = pl.ds(slot * 8, 8)
                    pltpu.async_copy(
                        ev_hbm.at[pl.ds(pl.multiple_of(jh[l] * 8, 8), 8), :],
                        e1m.at[dst], sem_h)
                    pltpu.async_copy(
                        rv_hbm.at[pl.ds(pl.multiple_of(jr[l] * 8, 8), 8), :],
                        erm.at[dst], sem_r)
                    pltpu.async_copy(
                        ev_hbm.at[pl.ds(pl.multiple_of(jt[l] * 8, 8), 8), :],
                        e2m.at[dst], sem_t)
            pltpu.make_async_copy(ev_hbm.at[pl.ds(0, slots)], e1m, sem_h).wait()
            pltpu.make_async_copy(rv_hbm.at[pl.ds(0, slots)], erm, sem_r).wait()
            pltpu.make_async_copy(ev_hbm.at[pl.ds(0, slots)], e2m, sem_t).wait()

            for b in range(p_items // 16):
                off = p * p_items + b * 16
                jh = idx_h[pl.ds(off, 16)]
                jr = idx_r[pl.ds(off, 16)]
                jt = idx_t[pl.ds(off, 16)]
                srow = (b * 16 + lane) * 8
                rh = srow + (jh & 7)
                rr = srow + (jr & 7)
                rt = srow + (jt & 7)
                acc = jnp.zeros((16,), jnp.float32)
                for c in range(D):
                    col = jnp.full((16,), c, jnp.int32)
                    h = plsc.load_gather(e1m, [rh, col])
                    r = plsc.load_gather(erm, [rr, col])
                    t = plsc.load_gather(e2m, [rt, col])
                    d = (h + r) - t
                    acc = acc + d * d
                outv[pl.ds(off, 16)] = -acc
            return carry

        lax.fori_loop(0, n_pass, do_pass, 0)

        pltpu.sync_copy(outv, out_hbm.at[pl.ds(base, b_per_w)])

    return k


def kernel(heads, relations, tails, entity_embedding, relation_embedding):
    B = heads.shape[0]
    D = entity_embedding.shape[1]
    k = _sc_transe(B, D, 32)
    return k(heads, relations, tails, entity_embedding, relation_embedding)


# speedup vs baseline: 1.4374x; 1.0867x over previous
import functools

import jax
import jax.numpy as jnp
from jax import lax
from jax.experimental import pallas as pl
from jax.experimental.pallas import tpu as pltpu
from jax.experimental.pallas import tpu_sc as plsc


def _sc_transe(B, D, n_workers):
    b_per_w = B // n_workers
    gsz = 16
    n_groups = b_per_w // gsz
    rows = gsz * 8
    mesh = plsc.VectorSubcoreMesh(core_axis_name="c", subcore_axis_name="s")
    num_cores = 2

    @functools.partial(
        pl.kernel,
        mesh=mesh,
        out_type=jax.ShapeDtypeStruct((B,), jnp.float32),
        compiler_params=pltpu.CompilerParams(
            needs_layout_passes=False, use_tc_tiling_on_sc=True),
        scratch_types=[
            pltpu.VMEM((b_per_w,), jnp.int32),
            pltpu.VMEM((b_per_w,), jnp.int32),
            pltpu.VMEM((b_per_w,), jnp.int32),
            pltpu.VMEM((rows, D), jnp.float32),
            pltpu.VMEM((rows, D), jnp.float32),
            pltpu.VMEM((rows, D), jnp.float32),
            pltpu.VMEM((rows, D), jnp.float32),
            pltpu.VMEM((rows, D), jnp.float32),
            pltpu.VMEM((rows, D), jnp.float32),
            pltpu.VMEM((b_per_w,), jnp.float32),
            pltpu.SemaphoreType.DMA,
            pltpu.SemaphoreType.DMA,
        ],
    )
    def k(heads_hbm, rel_hbm, tails_hbm, ev_hbm, rv_hbm, probe_hbm, out_hbm,
          idx_h, idx_r, idx_t, e1a, era, e2a, e1b, erb, e2b, outv,
          sem_a, sem_b):
        del probe_hbm
        wid = lax.axis_index("s") * num_cores + lax.axis_index("c")
        base = wid * b_per_w

        pltpu.sync_copy(heads_hbm.at[pl.ds(base, b_per_w)], idx_h)
        pltpu.sync_copy(rel_hbm.at[pl.ds(base, b_per_w)], idx_r)
        pltpu.sync_copy(tails_hbm.at[pl.ds(base, b_per_w)], idx_t)

        lane = lax.iota(jnp.int32, 16)

        def fire(g, e1m, erm, e2m, sem):
            off = g * gsz
            jh = idx_h[pl.ds(off, 16)] >> 3
            jr = idx_r[pl.ds(off, 16)] >> 3
            jt = idx_t[pl.ds(off, 16)] >> 3
            for l in range(16):
                dst = pl.ds(l * 8, 8)
                pltpu.async_copy(
                    ev_hbm.at[pl.ds(pl.multiple_of(jh[l] * 8, 8), 8), :],
                    e1m.at[dst], sem)
                pltpu.async_copy(
                    rv_hbm.at[pl.ds(pl.multiple_of(jr[l] * 8, 8), 8), :],
                    erm.at[dst], sem)
                pltpu.async_copy(
                    ev_hbm.at[pl.ds(pl.multiple_of(jt[l] * 8, 8), 8), :],
                    e2m.at[dst], sem)

        def drain(e1m, erm, e2m, sem):
            src = ev_hbm.at[pl.ds(0, rows), :]
            pltpu.make_async_copy(src, e1m, sem).wait()
            pltpu.make_async_copy(src, erm, sem).wait()
            pltpu.make_async_copy(src, e2m, sem).wait()

        def compute(g, e1m, erm, e2m):
            off = g * gsz
            srow = lane * 8
            rh = srow + (idx_h[pl.ds(off, 16)] & 7)
            rr = srow + (idx_r[pl.ds(off, 16)] & 7)
            rt = srow + (idx_t[pl.ds(off, 16)] & 7)
            acc = jnp.zeros((16,), jnp.float32)
            for c in range(D):
                col = jnp.full((16,), c, jnp.int32)
                h = plsc.load_gather(e1m, [rh, col])
                r = plsc.load_gather(erm, [rr, col])
                t = plsc.load_gather(e2m, [rt, col])
                d = (h + r) - t
                acc = acc + d * d
            outv[pl.ds(off, 16)] = -acc

        fire(0, e1a, era, e2a, sem_a)

        def body(i, carry):
            p = i * 2
            fire(p + 1, e1b, erb, e2b, sem_b)
            drain(e1a, era, e2a, sem_a)
            compute(p, e1a, era, e2a)

            @pl.when(i < n_groups // 2 - 1)
            def _():
                fire(p + 2, e1a, era, e2a, sem_a)

            drain(e1b, erb, e2b, sem_b)
            compute(p + 1, e1b, erb, e2b)
            return carry

        lax.fori_loop(0, n_groups // 2, body, 0)

        pltpu.sync_copy(outv, out_hbm.at[pl.ds(base, b_per_w)])

    return k


def kernel(heads, relations, tails, entity_embedding, relation_embedding):
    B = heads.shape[0]
    D = entity_embedding.shape[1]
    probe = jnp.take(entity_embedding, heads[:8], axis=0)
    k = _sc_transe(B, D, 32)
    return k(heads, relations, tails, entity_embedding, relation_embedding,
             probe)


# speedup vs baseline: 1.4495x; 1.0084x over previous
import functools

import jax
import jax.numpy as jnp
from jax import lax
from jax.experimental import pallas as pl
from jax.experimental.pallas import tpu as pltpu
from jax.experimental.pallas import tpu_sc as plsc


def _sc_transe(B, D, n_workers):
    b_per_w = B // n_workers
    gsz = 16
    n_groups = b_per_w // gsz
    rows = gsz * 8
    mesh = plsc.VectorSubcoreMesh(core_axis_name="c", subcore_axis_name="s")
    num_cores = 2

    @functools.partial(
        pl.kernel,
        mesh=mesh,
        out_type=jax.ShapeDtypeStruct((B,), jnp.float32),
        compiler_params=pltpu.CompilerParams(
            needs_layout_passes=False, use_tc_tiling_on_sc=True),
        scratch_types=[
            pltpu.VMEM((b_per_w,), jnp.int32),
            pltpu.VMEM((b_per_w,), jnp.int32),
            pltpu.VMEM((b_per_w,), jnp.int32),
            pltpu.VMEM((rows, D), jnp.float32),
            pltpu.VMEM((rows, D), jnp.float32),
            pltpu.VMEM((rows, D), jnp.float32),
            pltpu.VMEM((rows, D), jnp.float32),
            pltpu.VMEM((rows, D), jnp.float32),
            pltpu.VMEM((rows, D), jnp.float32),
            pltpu.VMEM((b_per_w,), jnp.float32),
            pltpu.SemaphoreType.DMA,
            pltpu.SemaphoreType.DMA,
        ],
    )
    def k(heads_hbm, rel_hbm, tails_hbm, ev_hbm, rv_hbm, out_hbm,
          idx_h, idx_r, idx_t, e1a, era, e2a, e1b, erb, e2b, outv,
          sem_a, sem_b):
        wid = lax.axis_index("s") * num_cores + lax.axis_index("c")
        base = wid * b_per_w

        pltpu.sync_copy(heads_hbm.at[pl.ds(base, b_per_w)], idx_h)
        pltpu.sync_copy(rel_hbm.at[pl.ds(base, b_per_w)], idx_r)
        pltpu.sync_copy(tails_hbm.at[pl.ds(base, b_per_w)], idx_t)

        lane = lax.iota(jnp.int32, 16)

        def fire(g, e1m, erm, e2m, sem):
            off = g * gsz
            jh = idx_h[pl.ds(off, 16)] >> 3
            jr = idx_r[pl.ds(off, 16)] >> 3
            jt = idx_t[pl.ds(off, 16)] >> 3
            for l in range(16):
                dst = pl.ds(l * 8, 8)
                pltpu.async_copy(
                    ev_hbm.at[pl.ds(pl.multiple_of(jh[l] * 8, 8), 8), :],
                    e1m.at[dst], sem)
                pltpu.async_copy(
                    rv_hbm.at[pl.ds(pl.multiple_of(jr[l] * 8, 8), 8), :],
                    erm.at[dst], sem)
                pltpu.async_copy(
                    ev_hbm.at[pl.ds(pl.multiple_of(jt[l] * 8, 8), 8), :],
                    e2m.at[dst], sem)

        def drain(e1m, erm, e2m, sem):
            src = ev_hbm.at[pl.ds(0, rows), :]
            pltpu.make_async_copy(src, e1m, sem).wait()
            pltpu.make_async_copy(src, erm, sem).wait()
            pltpu.make_async_copy(src, e2m, sem).wait()

        def compute(g, e1m, erm, e2m):
            off = g * gsz
            srow = lane * 8
            rh = srow + (idx_h[pl.ds(off, 16)] & 7)
            rr = srow + (idx_r[pl.ds(off, 16)] & 7)
            rt = srow + (idx_t[pl.ds(off, 16)] & 7)
            acc = jnp.zeros((16,), jnp.float32)
            for c in range(D):
                col = jnp.full((16,), c, jnp.int32)
                h = plsc.load_gather(e1m, [rh, col])
                r = plsc.load_gather(erm, [rr, col])
                t = plsc.load_gather(e2m, [rt, col])
                d = (h + r) - t
                acc = acc + d * d
            outv[pl.ds(off, 16)] = -acc

        fire(0, e1a, era, e2a, sem_a)

        def body(i, carry):
            p = i * 2
            fire(p + 1, e1b, erb, e2b, sem_b)
            drain(e1a, era, e2a, sem_a)
            compute(p, e1a, era, e2a)

            @pl.when(i < n_groups // 2 - 1)
            def _():
                fire(p + 2, e1a, era, e2a, sem_a)

            drain(e1b, erb, e2b, sem_b)
            compute(p + 1, e1b, erb, e2b)
            return carry

        lax.fori_loop(0, n_groups // 2, body, 0)

        pltpu.sync_copy(outv, out_hbm.at[pl.ds(base, b_per_w)])

    return k


def kernel(heads, relations, tails, entity_embedding, relation_embedding):
    B = heads.shape[0]
    D = entity_embedding.shape[1]
    k = _sc_transe(B, D, 32)
    return k(heads, relations, tails, entity_embedding, relation_embedding)


# speedup vs baseline: 1.5538x; 1.0719x over previous
import functools

import jax
import jax.numpy as jnp
from jax import lax
from jax.experimental import pallas as pl
from jax.experimental.pallas import tpu as pltpu
from jax.experimental.pallas import tpu_sc as plsc


def _sc_transe(B, D, n_workers):
    b_per_w = B // n_workers
    gsz = 16
    n_groups = b_per_w // gsz
    rows = gsz
    mesh = plsc.VectorSubcoreMesh(core_axis_name="c", subcore_axis_name="s")
    num_cores = 2

    @functools.partial(
        pl.kernel,
        mesh=mesh,
        out_type=jax.ShapeDtypeStruct((B,), jnp.float32),
        compiler_params=pltpu.CompilerParams(
            needs_layout_passes=False, use_tc_tiling_on_sc=True),
        scratch_types=[
            pltpu.VMEM((b_per_w,), jnp.int32),
            pltpu.VMEM((b_per_w,), jnp.int32),
            pltpu.VMEM((b_per_w,), jnp.int32),
            pltpu.VMEM((rows, D), jnp.float32),
            pltpu.VMEM((rows, D), jnp.float32),
            pltpu.VMEM((rows, D), jnp.float32),
            pltpu.VMEM((rows, D), jnp.float32),
            pltpu.VMEM((rows, D), jnp.float32),
            pltpu.VMEM((rows, D), jnp.float32),
            pltpu.VMEM((b_per_w,), jnp.float32),
            pltpu.SemaphoreType.DMA,
            pltpu.SemaphoreType.DMA,
        ],
    )
    def k(heads_hbm, rel_hbm, tails_hbm, ev_hbm, rv_hbm, out_hbm,
          idx_h, idx_r, idx_t, e1a, era, e2a, e1b, erb, e2b, outv,
          sem_a, sem_b):
        wid = lax.axis_index("s") * num_cores + lax.axis_index("c")
        base = wid * b_per_w

        pltpu.sync_copy(heads_hbm.at[pl.ds(base, b_per_w)], idx_h)
        pltpu.sync_copy(rel_hbm.at[pl.ds(base, b_per_w)], idx_r)
        pltpu.sync_copy(tails_hbm.at[pl.ds(base, b_per_w)], idx_t)

        lane = lax.iota(jnp.int32, 16)

        def fire(g, e1m, erm, e2m, sem):
            off = g * gsz
            jh = idx_h[pl.ds(off, 16)]
            jr = idx_r[pl.ds(off, 16)]
            jt = idx_t[pl.ds(off, 16)]
            for l in range(16):
                dst = pl.ds(l, 1)
                pltpu.async_copy(ev_hbm.at[pl.ds(jh[l], 1), :], e1m.at[dst], sem)
                pltpu.async_copy(rv_hbm.at[pl.ds(jr[l], 1), :], erm.at[dst], sem)
                pltpu.async_copy(ev_hbm.at[pl.ds(jt[l], 1), :], e2m.at[dst], sem)

        def drain(e1m, erm, e2m, sem):
            src = ev_hbm.at[pl.ds(0, rows), :]
            pltpu.make_async_copy(src, e1m, sem).wait()
            pltpu.make_async_copy(src, erm, sem).wait()
            pltpu.make_async_copy(src, e2m, sem).wait()

        def compute(g, e1m, erm, e2m):
            off = g * gsz
            rh = rr = rt = lane
            acc = jnp.zeros((16,), jnp.float32)
            for c in range(D):
                col = jnp.full((16,), c, jnp.int32)
                h = plsc.load_gather(e1m, [rh, col])
                r = plsc.load_gather(erm, [rr, col])
                t = plsc.load_gather(e2m, [rt, col])
                d = (h + r) - t
                acc = acc + d * d
            outv[pl.ds(off, 16)] = -acc

        fire(0, e1a, era, e2a, sem_a)

        def body(i, carry):
            p = i * 2
            fire(p + 1, e1b, erb, e2b, sem_b)
            drain(e1a, era, e2a, sem_a)
            compute(p, e1a, era, e2a)

            @pl.when(i < n_groups // 2 - 1)
            def _():
                fire(p + 2, e1a, era, e2a, sem_a)

            drain(e1b, erb, e2b, sem_b)
            compute(p + 1, e1b, erb, e2b)
            return carry

        lax.fori_loop(0, n_groups // 2, body, 0)

        pltpu.sync_copy(outv, out_hbm.at[pl.ds(base, b_per_w)])

    return k


def kernel(heads, relations, tails, entity_embedding, relation_embedding):
    B = heads.shape[0]
    D = entity_embedding.shape[1]
    k = _sc_transe(B, D, 32)
    return k(heads, relations, tails, entity_embedding, relation_embedding)
